# Initial kernel scaffold; baseline (speedup 1.0000x reference)
import jax
import jax.numpy as jnp
from jax.experimental import pallas as pl


def kernel(y_pred):
    raise NotImplementedError("write your pallas kernel here")


# speedup vs baseline: 1.7156x; 1.7156x over previous
import functools

import jax
import jax.numpy as jnp
from jax import lax
from jax.experimental import pallas as pl
from jax.experimental.pallas import tpu as pltpu

CLASS_NUM = 20
CONF_THRESH = 0.01
IOU_THRESHOLD = 0.45
TOP_K = 200
IMG_W = 512.0
BATCH = 8
N_BOXES = 20000
FEAT = 3 * CLASS_NUM + 5

ROWS = 160
LANES = 128
N_PAD = ROWS * LANES
NEG = -1e30


def _nms_body(yt_ref, out_ref, masked_ref, x1_ref, y1_ref, x2_ref, y2_ref,
              area_ref, cls_ref):
    yt = yt_ref[0]

    p1 = yt[CLASS_NUM:2 * CLASS_NUM]
    p2 = yt[2 * CLASS_NUM + 1:2 * CLASS_NUM + 1 + CLASS_NUM]
    prob = p1 * p2
    best = prob[0]
    bid = jnp.zeros((ROWS, LANES), jnp.float32)
    for c in range(1, CLASS_NUM):
        m = prob[c] > best
        best = jnp.where(m, prob[c], best)
        bid = jnp.where(m, jnp.float32(c), bid)

    def clip(v):
        return jnp.maximum(jnp.minimum(v, IMG_W - 1.0), 0.0)

    x1 = clip(yt[61])
    y1 = clip(yt[62])
    x2 = clip(yt[63])
    y2 = clip(yt[64])

    x1_ref[:] = x1
    y1_ref[:] = y1
    x2_ref[:] = x2
    y2_ref[:] = y2
    area_ref[:] = (jnp.maximum(x2 - x1, 0.0) * jnp.maximum(y2 - y1, 0.0))
    cls_ref[:] = bid + 1.0
    masked_ref[:] = jnp.where(best > CONF_THRESH, best, NEG)

    idx2d = (lax.broadcasted_iota(jnp.int32, (ROWS, LANES), 0) * LANES
             + lax.broadcasted_iota(jnp.int32, (ROWS, LANES), 1))
    lane = lax.broadcasted_iota(jnp.int32, (1, LANES), 1)

    def body(t, carry):
        m = masked_ref[:]
        v = jnp.max(m)
        has = v > jnp.float32(CONF_THRESH)
        idx = jnp.min(jnp.where(m == v, idx2d, jnp.int32(2 ** 30)))
        r = idx // LANES
        c = idx - r * LANES

        def pick(ref):
            return jnp.sum(jnp.where(lane == c, ref[pl.ds(r, 1), :], 0.0))

        sx1 = pick(x1_ref)
        sy1 = pick(y1_ref)
        sx2 = pick(x2_ref)
        sy2 = pick(y2_ref)
        scls = pick(cls_ref)
        sarea = (jnp.maximum(sx2 - sx1, 0.0) * jnp.maximum(sy2 - sy1, 0.0))

        xx1 = jnp.maximum(sx1, x1_ref[:])
        yy1 = jnp.maximum(sy1, y1_ref[:])
        xx2 = jnp.minimum(sx2, x2_ref[:])
        yy2 = jnp.minimum(sy2, y2_ref[:])
        inter = (jnp.maximum(xx2 - xx1, 0.0) * jnp.maximum(yy2 - yy1, 0.0))
        union = sarea + area_ref[:] - inter
        sup = (union > 0.0) & (inter > IOU_THRESHOLD * jnp.maximum(union, 1e-12))
        newm = jnp.where(sup | (idx2d == idx), NEG, m)
        masked_ref[:] = jnp.where(has, newm, m)

        row = jnp.zeros((1, LANES), jnp.float32)
        for j, val in enumerate((scls, v, sx1, sy1, sx2, sy2)):
            row = jnp.where(lane == j, val, row)
        row = jnp.where(has, row, 0.0)
        out_ref[0, pl.ds(t, 1), :] = row
        return carry

    lax.fori_loop(0, TOP_K, body, 0, unroll=False)


@functools.partial(jax.jit, static_argnames=("interpret",))
def kernel(y_pred, interpret=False):
    yp = jnp.pad(y_pred, ((0, 0), (0, N_PAD - N_BOXES), (0, 0)))
    yt = jnp.transpose(yp, (0, 2, 1)).reshape(BATCH, FEAT, ROWS, LANES)

    out = pl.pallas_call(
        _nms_body,
        grid=(BATCH,),
        in_specs=[pl.BlockSpec((1, FEAT, ROWS, LANES), lambda b: (b, 0, 0, 0))],
        out_specs=pl.BlockSpec((1, TOP_K, LANES), lambda b: (b, 0, 0)),
        out_shape=jax.ShapeDtypeStruct((BATCH, TOP_K, LANES), jnp.float32),
        scratch_shapes=[pltpu.VMEM((ROWS, LANES), jnp.float32)
                        for _ in range(7)],
        interpret=interpret,
    )(yt)
    return out[:, :, :6]


# speedup vs baseline: 3.0632x; 1.7855x over previous
import functools

import jax
import jax.numpy as jnp
from jax import lax
from jax.experimental import pallas as pl
from jax.experimental.pallas import tpu as pltpu
from jax.experimental.pallas import tpu_sc as plsc

CLASS_NUM = 20
CONF_THRESH = 0.01
IOU_THRESHOLD = 0.45
TOP_K = 200
IMG_W = 512.0
BATCH = 8
N_BOXES = 20000
FEAT = 3 * CLASS_NUM + 5

ROWS = 160
LANES = 128
N_PAD = ROWS * LANES
NEG = -1e30

RB = 32

L = 16
NV = N_PAD // L
NG1 = NV // L
NG2 = NG1 // L
KEPT_PAD = 224
OUT_W = 8
OUT_FLAT = TOP_K * OUT_W


def _decode_body(y_ref, conf_ref, cls_ref, x1_ref, y1_ref, x2_ref, y2_ref):
    yt = y_ref[0]
    p1 = yt[:, :, CLASS_NUM:2 * CLASS_NUM]
    p2 = yt[:, :, 2 * CLASS_NUM + 1:3 * CLASS_NUM + 1]
    prob = p1 * p2
    conf = jnp.max(prob, axis=2)
    iota = lax.broadcasted_iota(jnp.int32, prob.shape, 2)
    bid = jnp.min(jnp.where(prob == conf[:, :, None], iota, CLASS_NUM), axis=2)

    def clip(v):
        return jnp.maximum(jnp.minimum(v, IMG_W - 1.0), 0.0)

    conf_ref[0] = conf
    cls_ref[0] = bid.astype(jnp.float32) + 1.0
    x1_ref[0] = clip(yt[:, :, 61])
    y1_ref[0] = clip(yt[:, :, 62])
    x2_ref[0] = clip(yt[:, :, 63])
    y2_ref[0] = clip(yt[:, :, 64])


def _sc_nms_body(conf_hbm, cls_hbm, x1_hbm, y1_hbm, x2_hbm, y2_hbm, out_hbm,
                 conf_v, cls_v, x1_v, y1_v, x2_v, y2_v, l1_v, l2_v,
                 kx1_v, ky1_v, kx2_v, ky2_v, kar_v, out_v):
    wid = lax.axis_index("s") * 2 + lax.axis_index("c")

    @pl.when(wid < BATCH)
    def _run():
        b = wid
        pltpu.sync_copy(conf_hbm.at[b], conf_v)
        pltpu.sync_copy(cls_hbm.at[b], cls_v)
        pltpu.sync_copy(x1_hbm.at[b], x1_v)
        pltpu.sync_copy(y1_hbm.at[b], y1_v)
        pltpu.sync_copy(x2_hbm.at[b], x2_v)
        pltpu.sync_copy(y2_hbm.at[b], y2_v)

        iota = jnp.arange(L, dtype=jnp.int32)
        zf = jnp.zeros((L,), jnp.float32)
        lane0 = iota == 0

        def _zero(i, _):
            out_v[pl.ds(i * L, L)] = zf
            return 0
        lax.fori_loop(0, OUT_FLAT // L, _zero, 0)
        for w in range(KEPT_PAD // L):
            kx1_v[pl.ds(w * L, L)] = zf
            ky1_v[pl.ds(w * L, L)] = zf
            kx2_v[pl.ds(w * L, L)] = zf
            ky2_v[pl.ds(w * L, L)] = zf
            kar_v[pl.ds(w * L, L)] = zf

        negv = jnp.full((L,), NEG, jnp.float32)

        def _build_l1(g, _):
            acc = negv
            base = g * (L * L)
            for k in range(L):
                v = conf_v[pl.ds(base + k * L, L)]
                v = jnp.where(v > CONF_THRESH, v, NEG)
                conf_v[pl.ds(base + k * L, L)] = v
                acc = jnp.maximum(acc, v)
            l1_v[pl.ds(g * L, L)] = acc
            return 0
        lax.fori_loop(0, NG1, _build_l1, 0)

        for h in range(NG2):
            acc = negv
            for k in range(L):
                acc = jnp.maximum(acc, l1_v[pl.ds((h * L + k) * L, L)])
            l2_v[pl.ds(h * L, L)] = acc

        def top_max():
            acc = l2_v[pl.ds(0, L)]
            for h in range(1, NG2):
                acc = jnp.maximum(acc, l2_v[pl.ds(h * L, L)])
            return jnp.max(acc)

        def cond(st):
            kcnt, m = st
            return (kcnt < TOP_K) & (m > CONF_THRESH)

        def body(st):
            kcnt, m = st
            msp = jnp.full((L,), m)

            hbest = jnp.full((L,), NG2, jnp.int32)
            for h in range(NG2):
                vec = l2_v[pl.ds(h * L, L)]
                hbest = jnp.minimum(hbest, jnp.where(vec == msp, h, NG2))
            hstar = jnp.min(hbest)

            kbest = jnp.full((L,), L, jnp.int32)
            for k in range(L):
                vec = l1_v[pl.ds((hstar * L + k) * L, L)]
                kbest = jnp.minimum(kbest, jnp.where(vec == msp, k, L))
            gstar = hstar * L + jnp.min(kbest)

            kbest2 = jnp.full((L,), L, jnp.int32)
            for k in range(L):
                vec = conf_v[pl.ds((gstar * L + k) * L, L)]
                kbest2 = jnp.minimum(kbest2, jnp.where(vec == msp, k, L))
            istar = gstar * L + jnp.min(kbest2)

            civ = conf_v[pl.ds(istar * L, L)]
            jstar = jnp.min(jnp.where(civ == msp, iota, L))
            bstar = istar * L + jstar
            bsp = jnp.full((L,), bstar, jnp.int32)

            sx1 = plsc.load_gather(x1_v, [bsp])
            sy1 = plsc.load_gather(y1_v, [bsp])
            sx2 = plsc.load_gather(x2_v, [bsp])
            sy2 = plsc.load_gather(y2_v, [bsp])
            scl = plsc.load_gather(cls_v, [bsp])
            sar = (jnp.maximum(sx2 - sx1, 0.0) * jnp.maximum(sy2 - sy1, 0.0))

            supm = jnp.zeros((L,), jnp.bool_)
            for w in range(KEPT_PAD // L):
                kx1 = kx1_v[pl.ds(w * L, L)]
                ky1 = ky1_v[pl.ds(w * L, L)]
                kx2 = kx2_v[pl.ds(w * L, L)]
                ky2 = ky2_v[pl.ds(w * L, L)]
                kar = kar_v[pl.ds(w * L, L)]
                xx1 = jnp.maximum(kx1, sx1)
                yy1 = jnp.maximum(ky1, sy1)
                xx2 = jnp.minimum(kx2, sx2)
                yy2 = jnp.minimum(ky2, sy2)
                inter = (jnp.maximum(xx2 - xx1, 0.0)
                         * jnp.maximum(yy2 - yy1, 0.0))
                union = kar + sar - inter
                s = (union > 0.0) & (inter > IOU_THRESHOLD
                                     * jnp.maximum(union, 1e-12))
                supm = supm | s
            keepb = jnp.max(supm.astype(jnp.int32)) == 0

            ksp = jnp.full((L,), kcnt, jnp.int32)
            appm = lane0 & keepb
            plsc.store_scatter(kx1_v, [ksp], sx1, mask=appm)
            plsc.store_scatter(ky1_v, [ksp], sy1, mask=appm)
            plsc.store_scatter(kx2_v, [ksp], sx2, mask=appm)
            plsc.store_scatter(ky2_v, [ksp], sy2, mask=appm)
            plsc.store_scatter(kar_v, [ksp], sar, mask=appm)

            row = jnp.where(iota == 0, scl,
                  jnp.where(iota == 1, msp,
                  jnp.where(iota == 2, sx1,
                  jnp.where(iota == 3, sy1,
                  jnp.where(iota == 4, sx2, sy2)))))
            owm = (iota < 6) & keepb
            plsc.store_scatter(out_v, [ksp * OUT_W + iota], row, mask=owm)

            plsc.store_scatter(conf_v, [bsp], negv, mask=lane0)
            l1slot = gstar * L + jstar
            gv = plsc.load_gather(conf_v, [(gstar * L + iota) * L + jstar])
            plsc.store_scatter(l1_v, [jnp.full((L,), l1slot, jnp.int32)],
                               jnp.full((L,), jnp.max(gv)), mask=lane0)
            l2slot = hstar * L + jstar
            hv = plsc.load_gather(l1_v, [(hstar * L + iota) * L + jstar])
            plsc.store_scatter(l2_v, [jnp.full((L,), l2slot, jnp.int32)],
                               jnp.full((L,), jnp.max(hv)), mask=lane0)

            return (kcnt + jnp.where(keepb, 1, 0), top_max())

        lax.while_loop(cond, body, (jnp.int32(0), top_max()))
        pltpu.sync_copy(out_v, out_hbm.at[b])


@jax.jit
def kernel(y_pred):
    yp = jnp.pad(y_pred, ((0, 0), (0, N_PAD - N_BOXES), (0, 0)))
    yp = yp.reshape(BATCH, ROWS, LANES, FEAT)

    shp = jax.ShapeDtypeStruct((BATCH, ROWS, LANES), jnp.float32)
    conf, cls, x1, y1, x2, y2 = pl.pallas_call(
        _decode_body,
        grid=(BATCH, ROWS // RB),
        in_specs=[pl.BlockSpec((1, RB, LANES, FEAT), lambda b, r: (b, r, 0, 0))],
        out_specs=[pl.BlockSpec((1, RB, LANES), lambda b, r: (b, r, 0))] * 6,
        out_shape=[shp] * 6,
    )(yp)

    nms = pl.kernel(
        _sc_nms_body,
        out_type=jax.ShapeDtypeStruct((BATCH, OUT_FLAT), jnp.float32),
        mesh=plsc.VectorSubcoreMesh(core_axis_name="c", subcore_axis_name="s",
                                    num_cores=2, num_subcores=16),
        compiler_params=pltpu.CompilerParams(needs_layout_passes=False),
        scratch_types=[
            pltpu.VMEM((N_PAD,), jnp.float32),
            pltpu.VMEM((N_PAD,), jnp.float32),
            pltpu.VMEM((N_PAD,), jnp.float32),
            pltpu.VMEM((N_PAD,), jnp.float32),
            pltpu.VMEM((N_PAD,), jnp.float32),
            pltpu.VMEM((N_PAD,), jnp.float32),
            pltpu.VMEM((NG1 * L,), jnp.float32),
            pltpu.VMEM((NG2 * L,), jnp.float32),
            pltpu.VMEM((KEPT_PAD,), jnp.float32),
            pltpu.VMEM((KEPT_PAD,), jnp.float32),
            pltpu.VMEM((KEPT_PAD,), jnp.float32),
            pltpu.VMEM((KEPT_PAD,), jnp.float32),
            pltpu.VMEM((KEPT_PAD,), jnp.float32),
            pltpu.VMEM((OUT_FLAT,), jnp.float32),
        ],
    )
    flat = lambda a: a.reshape(BATCH, N_PAD)
    out = nms(flat(conf), flat(cls), flat(x1), flat(y1), flat(x2), flat(y2))
    return out.reshape(BATCH, TOP_K, OUT_W)[:, :, :6]


# speedup vs baseline: 3.4234x; 1.1176x over previous
import jax
import jax.numpy as jnp
from jax import lax
from jax.experimental import pallas as pl
from jax.experimental.pallas import tpu as pltpu
from jax.experimental.pallas import tpu_sc as plsc

CLASS_NUM = 20
CONF_THRESH = 0.01
IOU_THRESHOLD = 0.45
TOP_K = 200
IMG_W = 512.0
BATCH = 8
N_BOXES = 20000
FEAT = 3 * CLASS_NUM + 5

ROWS = 160
LANES = 128
N_PAD = ROWS * LANES
NEG = -1e30

L = 16
NV = N_PAD // L
NG1 = NV // L
NG2 = NG1 // L
KEPT_PAD = 224
OUT_W = 8
OUT_FLAT = TOP_K * OUT_W


DEC_W = 32
WPB = DEC_W // BATCH
DEC_PER_W = N_PAD // WPB
DEC_CH = 160
DEC_NCH = DEC_PER_W // DEC_CH
DEC_NCH_LAST = (N_BOXES - (WPB - 1) * DEC_PER_W) // DEC_CH


def _sc_decode_body(y_hbm, conf_hbm, cls_hbm, x1_hbm, y1_hbm, x2_hbm, y2_hbm,
                    ybuf, cbuf, clbuf, x1buf, y1buf, x2buf, y2buf):
    wid = lax.axis_index("s") * 2 + lax.axis_index("c")
    b = wid // WPB
    part = wid - b * WPB
    n0 = part * DEC_PER_W
    iota = jnp.arange(L, dtype=jnp.int32)
    nch = jnp.where(part == WPB - 1, DEC_NCH_LAST, DEC_NCH)

    def chunk(ci, _):
        start = n0 + ci * DEC_CH
        pltpu.sync_copy(y_hbm.at[b, pl.ds(start, DEC_CH), :], ybuf)

        def feat(rows, f):
            return plsc.load_gather(ybuf, [rows, jnp.full((L,), f, jnp.int32)])

        for g in range(DEC_CH // L):
            rows = g * L + iota
            best = (feat(rows, CLASS_NUM) * feat(rows, 2 * CLASS_NUM + 1))
            bid = jnp.zeros((L,), jnp.float32)
            for c in range(1, CLASS_NUM):
                pc = (feat(rows, CLASS_NUM + c)
                      * feat(rows, 2 * CLASS_NUM + 1 + c))
                upd = pc > best
                best = jnp.where(upd, pc, best)
                bid = jnp.where(upd, jnp.float32(c), bid)

            def clipv(v):
                return jnp.maximum(jnp.minimum(v, IMG_W - 1.0), 0.0)

            sl = pl.ds(g * L, L)
            cbuf[sl] = best
            clbuf[sl] = bid + 1.0
            x1buf[sl] = clipv(feat(rows, 61))
            y1buf[sl] = clipv(feat(rows, 62))
            x2buf[sl] = clipv(feat(rows, 63))
            y2buf[sl] = clipv(feat(rows, 64))
        pltpu.sync_copy(cbuf, conf_hbm.at[b, pl.ds(start, DEC_CH)])
        pltpu.sync_copy(clbuf, cls_hbm.at[b, pl.ds(start, DEC_CH)])
        pltpu.sync_copy(x1buf, x1_hbm.at[b, pl.ds(start, DEC_CH)])
        pltpu.sync_copy(y1buf, y1_hbm.at[b, pl.ds(start, DEC_CH)])
        pltpu.sync_copy(x2buf, x2_hbm.at[b, pl.ds(start, DEC_CH)])
        pltpu.sync_copy(y2buf, y2_hbm.at[b, pl.ds(start, DEC_CH)])
        return 0

    lax.fori_loop(0, nch, chunk, 0)


def _sc_nms_body(conf_hbm, cls_hbm, x1_hbm, y1_hbm, x2_hbm, y2_hbm, out_hbm,
                 conf_v, cls_v, x1_v, y1_v, x2_v, y2_v, l1_v, l2_v,
                 kx1_v, ky1_v, kx2_v, ky2_v, kar_v, out_v):
    wid = lax.axis_index("s") * 2 + lax.axis_index("c")

    @pl.when(wid < BATCH)
    def _run():
        b = wid
        pltpu.sync_copy(conf_hbm.at[b], conf_v)
        pltpu.sync_copy(cls_hbm.at[b], cls_v)
        pltpu.sync_copy(x1_hbm.at[b], x1_v)
        pltpu.sync_copy(y1_hbm.at[b], y1_v)
        pltpu.sync_copy(x2_hbm.at[b], x2_v)
        pltpu.sync_copy(y2_hbm.at[b], y2_v)

        iota = jnp.arange(L, dtype=jnp.int32)
        zf = jnp.zeros((L,), jnp.float32)
        lane0 = iota == 0

        for i in range((N_PAD - N_BOXES) // L):
            conf_v[pl.ds(N_BOXES + i * L, L)] = jnp.full((L,), NEG, jnp.float32)

        def _zero(i, _):
            out_v[pl.ds(i * L, L)] = zf
            return 0
        lax.fori_loop(0, OUT_FLAT // L, _zero, 0)
        for w in range(KEPT_PAD // L):
            kx1_v[pl.ds(w * L, L)] = zf
            ky1_v[pl.ds(w * L, L)] = zf
            kx2_v[pl.ds(w * L, L)] = zf
            ky2_v[pl.ds(w * L, L)] = zf
            kar_v[pl.ds(w * L, L)] = zf

        negv = jnp.full((L,), NEG, jnp.float32)

        def _build_l1(g, _):
            acc = negv
            base = g * (L * L)
            for k in range(L):
                v = conf_v[pl.ds(base + k * L, L)]
                v = jnp.where(v > CONF_THRESH, v, NEG)
                conf_v[pl.ds(base + k * L, L)] = v
                acc = jnp.maximum(acc, v)
            l1_v[pl.ds(g * L, L)] = acc
            return 0
        lax.fori_loop(0, NG1, _build_l1, 0)

        for h in range(NG2):
            acc = negv
            for k in range(L):
                acc = jnp.maximum(acc, l1_v[pl.ds((h * L + k) * L, L)])
            l2_v[pl.ds(h * L, L)] = acc

        def top_max():
            acc = l2_v[pl.ds(0, L)]
            for h in range(1, NG2):
                acc = jnp.maximum(acc, l2_v[pl.ds(h * L, L)])
            return jnp.max(acc)

        def cond(st):
            kcnt, m = st
            return (kcnt < TOP_K) & (m > CONF_THRESH)

        def body(st):
            kcnt, m = st
            msp = jnp.full((L,), m)

            hbest = jnp.full((L,), NG2, jnp.int32)
            for h in range(NG2):
                vec = l2_v[pl.ds(h * L, L)]
                hbest = jnp.minimum(hbest, jnp.where(vec == msp, h, NG2))
            hstar = jnp.min(hbest)

            kbest = jnp.full((L,), L, jnp.int32)
            for k in range(L):
                vec = l1_v[pl.ds((hstar * L + k) * L, L)]
                kbest = jnp.minimum(kbest, jnp.where(vec == msp, k, L))
            gstar = hstar * L + jnp.min(kbest)

            kbest2 = jnp.full((L,), L, jnp.int32)
            for k in range(L):
                vec = conf_v[pl.ds((gstar * L + k) * L, L)]
                kbest2 = jnp.minimum(kbest2, jnp.where(vec == msp, k, L))
            istar = gstar * L + jnp.min(kbest2)

            civ = conf_v[pl.ds(istar * L, L)]
            jstar = jnp.min(jnp.where(civ == msp, iota, L))
            bstar = istar * L + jstar
            bsp = jnp.full((L,), bstar, jnp.int32)

            sx1 = plsc.load_gather(x1_v, [bsp])
            sy1 = plsc.load_gather(y1_v, [bsp])
            sx2 = plsc.load_gather(x2_v, [bsp])
            sy2 = plsc.load_gather(y2_v, [bsp])
            scl = plsc.load_gather(cls_v, [bsp])
            sar = (jnp.maximum(sx2 - sx1, 0.0) * jnp.maximum(sy2 - sy1, 0.0))

            supm = jnp.zeros((L,), jnp.bool_)
            for w in range(KEPT_PAD // L):
                kx1 = kx1_v[pl.ds(w * L, L)]
                ky1 = ky1_v[pl.ds(w * L, L)]
                kx2 = kx2_v[pl.ds(w * L, L)]
                ky2 = ky2_v[pl.ds(w * L, L)]
                kar = kar_v[pl.ds(w * L, L)]
                xx1 = jnp.maximum(kx1, sx1)
                yy1 = jnp.maximum(ky1, sy1)
                xx2 = jnp.minimum(kx2, sx2)
                yy2 = jnp.minimum(ky2, sy2)
                inter = (jnp.maximum(xx2 - xx1, 0.0)
                         * jnp.maximum(yy2 - yy1, 0.0))
                union = kar + sar - inter
                s = (union > 0.0) & (inter > IOU_THRESHOLD
                                     * jnp.maximum(union, 1e-12))
                supm = supm | s
            keepb = jnp.max(supm.astype(jnp.int32)) == 0

            ksp = jnp.full((L,), kcnt, jnp.int32)
            appm = lane0 & keepb
            plsc.store_scatter(kx1_v, [ksp], sx1, mask=appm)
            plsc.store_scatter(ky1_v, [ksp], sy1, mask=appm)
            plsc.store_scatter(kx2_v, [ksp], sx2, mask=appm)
            plsc.store_scatter(ky2_v, [ksp], sy2, mask=appm)
            plsc.store_scatter(kar_v, [ksp], sar, mask=appm)

            row = jnp.where(iota == 0, scl,
                  jnp.where(iota == 1, msp,
                  jnp.where(iota == 2, sx1,
                  jnp.where(iota == 3, sy1,
                  jnp.where(iota == 4, sx2, sy2)))))
            owm = (iota < 6) & keepb
            plsc.store_scatter(out_v, [ksp * OUT_W + iota], row, mask=owm)

            plsc.store_scatter(conf_v, [bsp], negv, mask=lane0)
            l1slot = gstar * L + jstar
            gv = plsc.load_gather(conf_v, [(gstar * L + iota) * L + jstar])
            plsc.store_scatter(l1_v, [jnp.full((L,), l1slot, jnp.int32)],
                               jnp.full((L,), jnp.max(gv)), mask=lane0)
            l2slot = hstar * L + jstar
            hv = plsc.load_gather(l1_v, [(hstar * L + iota) * L + jstar])
            plsc.store_scatter(l2_v, [jnp.full((L,), l2slot, jnp.int32)],
                               jnp.full((L,), jnp.max(hv)), mask=lane0)

            return (kcnt + jnp.where(keepb, 1, 0), top_max())

        lax.while_loop(cond, body, (jnp.int32(0), top_max()))
        pltpu.sync_copy(out_v, out_hbm.at[b])


@jax.jit
def kernel(y_pred):
    shp = jax.ShapeDtypeStruct((BATCH, N_PAD), jnp.float32)
    decode = pl.kernel(
        _sc_decode_body,
        out_type=[shp] * 6,
        mesh=plsc.VectorSubcoreMesh(core_axis_name="c", subcore_axis_name="s",
                                    num_cores=2, num_subcores=16),
        compiler_params=pltpu.CompilerParams(needs_layout_passes=False, use_tc_tiling_on_sc=False),
        scratch_types=[
            pltpu.VMEM((DEC_CH, FEAT), jnp.float32),
            pltpu.VMEM((DEC_CH,), jnp.float32),
            pltpu.VMEM((DEC_CH,), jnp.float32),
            pltpu.VMEM((DEC_CH,), jnp.float32),
            pltpu.VMEM((DEC_CH,), jnp.float32),
            pltpu.VMEM((DEC_CH,), jnp.float32),
            pltpu.VMEM((DEC_CH,), jnp.float32),
        ],
    )
    conf, cls, x1, y1, x2, y2 = decode(y_pred)

    nms = pl.kernel(
        _sc_nms_body,
        out_type=jax.ShapeDtypeStruct((BATCH, OUT_FLAT), jnp.float32),
        mesh=plsc.VectorSubcoreMesh(core_axis_name="c", subcore_axis_name="s",
                                    num_cores=2, num_subcores=16),
        compiler_params=pltpu.CompilerParams(needs_layout_passes=False, use_tc_tiling_on_sc=False),
        scratch_types=[
            pltpu.VMEM((N_PAD,), jnp.float32),
            pltpu.VMEM((N_PAD,), jnp.float32),
            pltpu.VMEM((N_PAD,), jnp.float32),
            pltpu.VMEM((N_PAD,), jnp.float32),
            pltpu.VMEM((N_PAD,), jnp.float32),
            pltpu.VMEM((N_PAD,), jnp.float32),
            pltpu.VMEM((NG1 * L,), jnp.float32),
            pltpu.VMEM((NG2 * L,), jnp.float32),
            pltpu.VMEM((KEPT_PAD,), jnp.float32),
            pltpu.VMEM((KEPT_PAD,), jnp.float32),
            pltpu.VMEM((KEPT_PAD,), jnp.float32),
            pltpu.VMEM((KEPT_PAD,), jnp.float32),
            pltpu.VMEM((KEPT_PAD,), jnp.float32),
            pltpu.VMEM((OUT_FLAT,), jnp.float32),
        ],
    )
    out = nms(conf, cls, x1, y1, x2, y2)
    return out.reshape(BATCH, TOP_K, OUT_W)[:, :, :6]


# speedup vs baseline: 4.0465x; 1.1820x over previous
import jax
import jax.numpy as jnp
from jax import lax
from jax.experimental import pallas as pl
from jax.experimental.pallas import tpu as pltpu
from jax.experimental.pallas import tpu_sc as plsc

CLASS_NUM = 20
CONF_THRESH = 0.01
IOU_THRESHOLD = 0.45
TOP_K = 200
IMG_W = 512.0
BATCH = 8
N_BOXES = 20000
FEAT = 3 * CLASS_NUM + 5

N_PAD = 20480
NEG = -1e30

L = 16
NG1 = N_PAD // (L * L)
NG2 = NG1 // L
KEPT_PAD = 224
OUT_W = 8
OUT_FLAT = TOP_K * OUT_W

BPC = BATCH // 2
WPB = 4
DEC_PER_W = N_PAD // WPB
DEC_CH = 160
DEC_NCH = DEC_PER_W // DEC_CH
DEC_NCH_LAST = (N_BOXES - (WPB - 1) * DEC_PER_W) // DEC_CH
YB0 = 0
ST0 = DEC_CH * FEAT


def _sc_body(y_hbm, out_hbm, dec_hbm,
             conf_v, cls_v, x1_v, y1_v, x2_v, y2_v, l1_v, l2_v,
             kx1_v, ky1_v, kx2_v, ky2_v, kar_v, out_v):
    c = lax.axis_index("c")
    s = lax.axis_index("s")
    iota = jnp.arange(L, dtype=jnp.int32)

    lb = s // WPB
    q = s - lb * WPB
    b = BPC * c + lb
    n0 = q * DEC_PER_W
    nch = jnp.where(q == WPB - 1, DEC_NCH_LAST, DEC_NCH)

    def chunk(ci, _):
        start = n0 + ci * DEC_CH
        pltpu.sync_copy(y_hbm.at[b, pl.ds(start * FEAT, DEC_CH * FEAT)],
                        cls_v.at[pl.ds(YB0, DEC_CH * FEAT)])

        def feat(rows, f):
            return plsc.load_gather(cls_v, [YB0 + rows * FEAT + f])

        for g in range(DEC_CH // L):
            rows = g * L + iota
            best = feat(rows, CLASS_NUM) * feat(rows, 2 * CLASS_NUM + 1)
            bid = jnp.zeros((L,), jnp.float32)
            for cc in range(1, CLASS_NUM):
                pc = (feat(rows, CLASS_NUM + cc)
                      * feat(rows, 2 * CLASS_NUM + 1 + cc))
                upd = pc > best
                best = jnp.where(upd, pc, best)
                bid = jnp.where(upd, jnp.float32(cc), bid)

            def clipv(v):
                return jnp.maximum(jnp.minimum(v, IMG_W - 1.0), 0.0)

            o = ST0 + g * L
            cls_v[pl.ds(o, L)] = best
            cls_v[pl.ds(o + DEC_CH, L)] = bid + 1.0
            cls_v[pl.ds(o + 2 * DEC_CH, L)] = clipv(feat(rows, 61))
            cls_v[pl.ds(o + 3 * DEC_CH, L)] = clipv(feat(rows, 62))
            cls_v[pl.ds(o + 4 * DEC_CH, L)] = clipv(feat(rows, 63))
            cls_v[pl.ds(o + 5 * DEC_CH, L)] = clipv(feat(rows, 64))

        for a in range(6):
            pltpu.sync_copy(cls_v.at[pl.ds(ST0 + a * DEC_CH, DEC_CH)],
                            dec_hbm.at[a, b, pl.ds(start, DEC_CH)])
        return 0

    lax.fori_loop(0, nch, chunk, 0)
    plsc.subcore_barrier()

    @pl.when(s < BPC)
    def _run():
        gb = BPC * c + s
        for a, dstv in enumerate((conf_v, cls_v, x1_v, y1_v, x2_v, y2_v)):
            pltpu.sync_copy(dec_hbm.at[a, gb], dstv)

        zf = jnp.zeros((L,), jnp.float32)
        lane0 = iota == 0
        negv = jnp.full((L,), NEG, jnp.float32)

        for i in range((N_PAD - N_BOXES) // L):
            conf_v[pl.ds(N_BOXES + i * L, L)] = negv

        def _zero(i, _):
            out_v[pl.ds(i * L, L)] = zf
            return 0
        lax.fori_loop(0, OUT_FLAT // L, _zero, 0)
        for w in range(KEPT_PAD // L):
            kx1_v[pl.ds(w * L, L)] = zf
            ky1_v[pl.ds(w * L, L)] = zf
            kx2_v[pl.ds(w * L, L)] = zf
            ky2_v[pl.ds(w * L, L)] = zf
            kar_v[pl.ds(w * L, L)] = zf

        def _build_l1(g, _):
            acc = negv
            base = g * (L * L)
            for k in range(L):
                v = conf_v[pl.ds(base + k * L, L)]
                v = jnp.where(v > CONF_THRESH, v, NEG)
                conf_v[pl.ds(base + k * L, L)] = v
                acc = jnp.maximum(acc, v)
            l1_v[pl.ds(g * L, L)] = acc
            return 0
        lax.fori_loop(0, NG1, _build_l1, 0)

        for h in range(NG2):
            acc = negv
            for k in range(L):
                acc = jnp.maximum(acc, l1_v[pl.ds((h * L + k) * L, L)])
            l2_v[pl.ds(h * L, L)] = acc

        def top_max():
            acc = l2_v[pl.ds(0, L)]
            for h in range(1, NG2):
                acc = jnp.maximum(acc, l2_v[pl.ds(h * L, L)])
            return jnp.max(acc)

        def cond(st):
            kcnt, m = st
            return (kcnt < TOP_K) & (m > CONF_THRESH)

        def body(st):
            kcnt, m = st
            msp = jnp.full((L,), m)

            hbest = jnp.full((L,), NG2, jnp.int32)
            for h in range(NG2):
                vec = l2_v[pl.ds(h * L, L)]
                hbest = jnp.minimum(hbest, jnp.where(vec == msp, h, NG2))
            hstar = jnp.min(hbest)

            kbest = jnp.full((L,), L, jnp.int32)
            for k in range(L):
                vec = l1_v[pl.ds((hstar * L + k) * L, L)]
                kbest = jnp.minimum(kbest, jnp.where(vec == msp, k, L))
            gstar = hstar * L + jnp.min(kbest)

            kbest2 = jnp.full((L,), L, jnp.int32)
            for k in range(L):
                vec = conf_v[pl.ds((gstar * L + k) * L, L)]
                kbest2 = jnp.minimum(kbest2, jnp.where(vec == msp, k, L))
            istar = gstar * L + jnp.min(kbest2)

            civ = conf_v[pl.ds(istar * L, L)]
            jstar = jnp.min(jnp.where(civ == msp, iota, L))
            bstar = istar * L + jstar
            bsp = jnp.full((L,), bstar, jnp.int32)

            sx1 = plsc.load_gather(x1_v, [bsp])
            sy1 = plsc.load_gather(y1_v, [bsp])
            sx2 = plsc.load_gather(x2_v, [bsp])
            sy2 = plsc.load_gather(y2_v, [bsp])
            scl = plsc.load_gather(cls_v, [bsp])
            sar = (jnp.maximum(sx2 - sx1, 0.0) * jnp.maximum(sy2 - sy1, 0.0))

            supm = jnp.zeros((L,), jnp.bool_)
            for w in range(KEPT_PAD // L):
                kx1 = kx1_v[pl.ds(w * L, L)]
                ky1 = ky1_v[pl.ds(w * L, L)]
                kx2 = kx2_v[pl.ds(w * L, L)]
                ky2 = ky2_v[pl.ds(w * L, L)]
                kar = kar_v[pl.ds(w * L, L)]
                xx1 = jnp.maximum(kx1, sx1)
                yy1 = jnp.maximum(ky1, sy1)
                xx2 = jnp.minimum(kx2, sx2)
                yy2 = jnp.minimum(ky2, sy2)
                inter = (jnp.maximum(xx2 - xx1, 0.0)
                         * jnp.maximum(yy2 - yy1, 0.0))
                union = kar + sar - inter
                sv = (union > 0.0) & (inter > IOU_THRESHOLD
                                      * jnp.maximum(union, 1e-12))
                supm = supm | sv
            keepb = jnp.max(supm.astype(jnp.int32)) == 0

            ksp = jnp.full((L,), kcnt, jnp.int32)
            appm = lane0 & keepb
            plsc.store_scatter(kx1_v, [ksp], sx1, mask=appm)
            plsc.store_scatter(ky1_v, [ksp], sy1, mask=appm)
            plsc.store_scatter(kx2_v, [ksp], sx2, mask=appm)
            plsc.store_scatter(ky2_v, [ksp], sy2, mask=appm)
            plsc.store_scatter(kar_v, [ksp], sar, mask=appm)

            row = jnp.where(iota == 0, scl,
                  jnp.where(iota == 1, msp,
                  jnp.where(iota == 2, sx1,
                  jnp.where(iota == 3, sy1,
                  jnp.where(iota == 4, sx2, sy2)))))
            owm = (iota < 6) & keepb
            plsc.store_scatter(out_v, [ksp * OUT_W + iota], row, mask=owm)

            plsc.store_scatter(conf_v, [bsp], negv, mask=lane0)
            l1slot = gstar * L + jstar
            gv = plsc.load_gather(conf_v, [(gstar * L + iota) * L + jstar])
            plsc.store_scatter(l1_v, [jnp.full((L,), l1slot, jnp.int32)],
                               jnp.full((L,), jnp.max(gv)), mask=lane0)
            l2slot = hstar * L + jstar
            hv = plsc.load_gather(l1_v, [(hstar * L + iota) * L + jstar])
            plsc.store_scatter(l2_v, [jnp.full((L,), l2slot, jnp.int32)],
                               jnp.full((L,), jnp.max(hv)), mask=lane0)

            return (kcnt + jnp.where(keepb, 1, 0), top_max())

        lax.while_loop(cond, body, (jnp.int32(0), top_max()))
        pltpu.sync_copy(out_v, out_hbm.at[BPC * c + s])


@jax.jit
def kernel(y_pred):
    run = pl.kernel(
        _sc_body,
        out_type=[jax.ShapeDtypeStruct((BATCH, OUT_FLAT), jnp.float32),
                  jax.ShapeDtypeStruct((6, BATCH, N_PAD), jnp.float32)],
        mesh=plsc.VectorSubcoreMesh(core_axis_name="c", subcore_axis_name="s",
                                    num_cores=2, num_subcores=16),
        compiler_params=pltpu.CompilerParams(needs_layout_passes=False,
                                             use_tc_tiling_on_sc=False),
        scratch_types=[
            pltpu.VMEM((N_PAD,), jnp.float32),
            pltpu.VMEM((N_PAD,), jnp.float32),
            pltpu.VMEM((N_PAD,), jnp.float32),
            pltpu.VMEM((N_PAD,), jnp.float32),
            pltpu.VMEM((N_PAD,), jnp.float32),
            pltpu.VMEM((N_PAD,), jnp.float32),
            pltpu.VMEM((NG1 * L,), jnp.float32),
            pltpu.VMEM((NG2 * L,), jnp.float32),
            pltpu.VMEM((KEPT_PAD,), jnp.float32),
            pltpu.VMEM((KEPT_PAD,), jnp.float32),
            pltpu.VMEM((KEPT_PAD,), jnp.float32),
            pltpu.VMEM((KEPT_PAD,), jnp.float32),
            pltpu.VMEM((KEPT_PAD,), jnp.float32),
            pltpu.VMEM((OUT_FLAT,), jnp.float32),
        ],
    )
    out, _ = run(y_pred.reshape(BATCH, N_BOXES * FEAT))
    return out.reshape(BATCH, TOP_K, OUT_W)[:, :, :6]


# speedup vs baseline: 4.3446x; 1.0737x over previous
import jax
import jax.numpy as jnp
from jax import lax
from jax.experimental import pallas as pl
from jax.experimental.pallas import tpu as pltpu
from jax.experimental.pallas import tpu_sc as plsc

CLASS_NUM = 20
CONF_THRESH = 0.01
IOU_THRESHOLD = 0.45
TOP_K = 200
IMG_W = 512.0
BATCH = 8
N_BOXES = 20000
FEAT = 3 * CLASS_NUM + 5

N_PAD = 20480
NEG = -1e30

L = 16
NG1 = N_PAD // (L * L)
NG2 = NG1 // L
KEPT_PAD = 224
OUT_W = 8
OUT_FLAT = TOP_K * OUT_W

BPC = BATCH // 2
WPB = 4
DEC_PER_W = N_PAD // WPB
DEC_CH = 160
DEC_NCH = DEC_PER_W // DEC_CH
DEC_NCH_LAST = (N_BOXES - (WPB - 1) * DEC_PER_W) // DEC_CH
YB0 = 0
ST0 = DEC_CH * FEAT


def _sc_body(y_hbm, out_hbm, dec_hbm,
             conf_v, cls_v, x1_v, y1_v, x2_v, y2_v, l1_v, l2_v,
             kx1_v, ky1_v, kx2_v, ky2_v, kar_v, out_v, dsemA, dsemB, ssem):
    c = lax.axis_index("c")
    s = lax.axis_index("s")
    iota = jnp.arange(L, dtype=jnp.int32)

    lb = s // WPB
    q = s - lb * WPB
    b = BPC * c + lb
    n0 = q * DEC_PER_W
    nch = jnp.where(q == WPB - 1, DEC_NCH_LAST, DEC_NCH)

    CHW = DEC_CH * FEAT

    def fire(ci, yref, base, sem):
        start = n0 + ci * DEC_CH
        pltpu.make_async_copy(y_hbm.at[b, pl.ds(start * FEAT, CHW)],
                              yref.at[pl.ds(base, CHW)], sem).start()

    def compute(ci, yref, base):
        start = n0 + ci * DEC_CH

        def feat(rows, f):
            return plsc.load_gather(yref, [base + rows * FEAT + f])

        for g in range(DEC_CH // L):
            rows = g * L + iota
            best = feat(rows, CLASS_NUM) * feat(rows, 2 * CLASS_NUM + 1)
            bid = jnp.zeros((L,), jnp.float32)
            for cc in range(1, CLASS_NUM):
                pc = (feat(rows, CLASS_NUM + cc)
                      * feat(rows, 2 * CLASS_NUM + 1 + cc))
                upd = pc > best
                best = jnp.where(upd, pc, best)
                bid = jnp.where(upd, jnp.float32(cc), bid)

            def clipv(v):
                return jnp.maximum(jnp.minimum(v, IMG_W - 1.0), 0.0)

            o = ST0 + g * L
            x1_v[pl.ds(o, L)] = best
            x1_v[pl.ds(o + DEC_CH, L)] = bid + 1.0
            x1_v[pl.ds(o + 2 * DEC_CH, L)] = clipv(feat(rows, 61))
            x1_v[pl.ds(o + 3 * DEC_CH, L)] = clipv(feat(rows, 62))
            x1_v[pl.ds(o + 4 * DEC_CH, L)] = clipv(feat(rows, 63))
            x1_v[pl.ds(o + 5 * DEC_CH, L)] = clipv(feat(rows, 64))

        for a in range(6):
            pltpu.sync_copy(x1_v.at[pl.ds(ST0 + a * DEC_CH, DEC_CH)],
                            dec_hbm.at[a, b, pl.ds(start, DEC_CH)])

    fire(0, cls_v, 0, dsemA)

    def chunk(ci, _):
        def even():
            start = n0 + ci * DEC_CH
            pltpu.make_async_copy(y_hbm.at[b, pl.ds(start * FEAT, CHW)],
                                  cls_v.at[pl.ds(0, CHW)], dsemA).wait()

            @pl.when(ci + 1 < nch)
            def _f():
                fire(ci + 1, y1_v, 0, dsemB)
            compute(ci, cls_v, 0)

        def odd():
            start = n0 + ci * DEC_CH
            pltpu.make_async_copy(y_hbm.at[b, pl.ds(start * FEAT, CHW)],
                                  y1_v.at[pl.ds(0, CHW)], dsemB).wait()

            @pl.when(ci + 1 < nch)
            def _f():
                fire(ci + 1, cls_v, 0, dsemA)
            compute(ci, y1_v, 0)

        lax.cond(ci % 2 == 0, even, odd)
        return 0

    lax.fori_loop(0, nch, chunk, 0)
    plsc.subcore_barrier()

    @pl.when(s < BPC)
    def _run():
        gb = BPC * c + s
        arrs = (conf_v, cls_v, x1_v, y1_v, x2_v, y2_v)
        for a, dstv in enumerate(arrs):
            pltpu.make_async_copy(dec_hbm.at[a, gb], dstv, ssem).start()
        for a, dstv in enumerate(arrs):
            pltpu.make_async_copy(dec_hbm.at[a, gb], dstv, ssem).wait()

        zf = jnp.zeros((L,), jnp.float32)
        lane0 = iota == 0
        negv = jnp.full((L,), NEG, jnp.float32)

        for i in range((N_PAD - N_BOXES) // L):
            conf_v[pl.ds(N_BOXES + i * L, L)] = negv

        def _zero(i, _):
            out_v[pl.ds(i * L, L)] = zf
            return 0
        lax.fori_loop(0, OUT_FLAT // L, _zero, 0)
        for w in range(KEPT_PAD // L):
            kx1_v[pl.ds(w * L, L)] = zf
            ky1_v[pl.ds(w * L, L)] = zf
            kx2_v[pl.ds(w * L, L)] = zf
            ky2_v[pl.ds(w * L, L)] = zf
            kar_v[pl.ds(w * L, L)] = zf

        def _build_l1(g, _):
            acc = negv
            base = g * (L * L)
            for k in range(L):
                v = conf_v[pl.ds(base + k * L, L)]
                v = jnp.where(v > CONF_THRESH, v, NEG)
                conf_v[pl.ds(base + k * L, L)] = v
                acc = jnp.maximum(acc, v)
            l1_v[pl.ds(g * L, L)] = acc
            return 0
        lax.fori_loop(0, NG1, _build_l1, 0)

        for h in range(NG2):
            acc = negv
            for k in range(L):
                acc = jnp.maximum(acc, l1_v[pl.ds((h * L + k) * L, L)])
            l2_v[pl.ds(h * L, L)] = acc

        def top_max():
            acc = l2_v[pl.ds(0, L)]
            for h in range(1, NG2):
                acc = jnp.maximum(acc, l2_v[pl.ds(h * L, L)])
            return jnp.max(acc)

        def cond(st):
            kcnt, m = st
            return (kcnt < TOP_K) & (m > CONF_THRESH)

        def body(st):
            kcnt, m = st
            msp = jnp.full((L,), m)

            hbest = jnp.full((L,), NG2, jnp.int32)
            for h in range(NG2):
                vec = l2_v[pl.ds(h * L, L)]
                hbest = jnp.minimum(hbest, jnp.where(vec == msp, h, NG2))
            hstar = jnp.min(hbest)

            kbest = jnp.full((L,), L, jnp.int32)
            for k in range(L):
                vec = l1_v[pl.ds((hstar * L + k) * L, L)]
                kbest = jnp.minimum(kbest, jnp.where(vec == msp, k, L))
            gstar = hstar * L + jnp.min(kbest)

            kbest2 = jnp.full((L,), L, jnp.int32)
            for k in range(L):
                vec = conf_v[pl.ds((gstar * L + k) * L, L)]
                kbest2 = jnp.minimum(kbest2, jnp.where(vec == msp, k, L))
            istar = gstar * L + jnp.min(kbest2)

            civ = conf_v[pl.ds(istar * L, L)]
            jstar = jnp.min(jnp.where(civ == msp, iota, L))
            bstar = istar * L + jstar
            bsp = jnp.full((L,), bstar, jnp.int32)

            sx1 = plsc.load_gather(x1_v, [bsp])
            sy1 = plsc.load_gather(y1_v, [bsp])
            sx2 = plsc.load_gather(x2_v, [bsp])
            sy2 = plsc.load_gather(y2_v, [bsp])
            scl = plsc.load_gather(cls_v, [bsp])
            sar = (jnp.maximum(sx2 - sx1, 0.0) * jnp.maximum(sy2 - sy1, 0.0))

            supm = jnp.zeros((L,), jnp.bool_)
            for w in range(KEPT_PAD // L):
                kx1 = kx1_v[pl.ds(w * L, L)]
                ky1 = ky1_v[pl.ds(w * L, L)]
                kx2 = kx2_v[pl.ds(w * L, L)]
                ky2 = ky2_v[pl.ds(w * L, L)]
                kar = kar_v[pl.ds(w * L, L)]
                xx1 = jnp.maximum(kx1, sx1)
                yy1 = jnp.maximum(ky1, sy1)
                xx2 = jnp.minimum(kx2, sx2)
                yy2 = jnp.minimum(ky2, sy2)
                inter = (jnp.maximum(xx2 - xx1, 0.0)
                         * jnp.maximum(yy2 - yy1, 0.0))
                union = kar + sar - inter
                sv = (union > 0.0) & (inter > IOU_THRESHOLD
                                      * jnp.maximum(union, 1e-12))
                supm = supm | sv
            keepb = jnp.max(supm.astype(jnp.int32)) == 0

            ksp = jnp.full((L,), kcnt, jnp.int32)
            appm = lane0 & keepb
            plsc.store_scatter(kx1_v, [ksp], sx1, mask=appm)
            plsc.store_scatter(ky1_v, [ksp], sy1, mask=appm)
            plsc.store_scatter(kx2_v, [ksp], sx2, mask=appm)
            plsc.store_scatter(ky2_v, [ksp], sy2, mask=appm)
            plsc.store_scatter(kar_v, [ksp], sar, mask=appm)

            row = jnp.where(iota == 0, scl,
                  jnp.where(iota == 1, msp,
                  jnp.where(iota == 2, sx1,
                  jnp.where(iota == 3, sy1,
                  jnp.where(iota == 4, sx2, sy2)))))
            owm = (iota < 6) & keepb
            plsc.store_scatter(out_v, [ksp * OUT_W + iota], row, mask=owm)

            plsc.store_scatter(conf_v, [bsp], negv, mask=lane0)
            l1slot = gstar * L + jstar
            gv = plsc.load_gather(conf_v, [(gstar * L + iota) * L + jstar])
            plsc.store_scatter(l1_v, [jnp.full((L,), l1slot, jnp.int32)],
                               jnp.full((L,), jnp.max(gv)), mask=lane0)
            l2slot = hstar * L + jstar
            hv = plsc.load_gather(l1_v, [(hstar * L + iota) * L + jstar])
            plsc.store_scatter(l2_v, [jnp.full((L,), l2slot, jnp.int32)],
                               jnp.full((L,), jnp.max(hv)), mask=lane0)

            return (kcnt + jnp.where(keepb, 1, 0), top_max())

        lax.while_loop(cond, body, (jnp.int32(0), top_max()))
        pltpu.sync_copy(out_v, out_hbm.at[BPC * c + s])


@jax.jit
def kernel(y_pred):
    run = pl.kernel(
        _sc_body,
        out_type=[jax.ShapeDtypeStruct((BATCH, OUT_FLAT), jnp.float32),
                  jax.ShapeDtypeStruct((6, BATCH, N_PAD), jnp.float32)],
        mesh=plsc.VectorSubcoreMesh(core_axis_name="c", subcore_axis_name="s",
                                    num_cores=2, num_subcores=16),
        compiler_params=pltpu.CompilerParams(needs_layout_passes=False,
                                             use_tc_tiling_on_sc=False),
        scratch_types=[
            pltpu.VMEM((N_PAD,), jnp.float32),
            pltpu.VMEM((N_PAD,), jnp.float32),
            pltpu.VMEM((N_PAD,), jnp.float32),
            pltpu.VMEM((N_PAD,), jnp.float32),
            pltpu.VMEM((N_PAD,), jnp.float32),
            pltpu.VMEM((N_PAD,), jnp.float32),
            pltpu.VMEM((NG1 * L,), jnp.float32),
            pltpu.VMEM((NG2 * L,), jnp.float32),
            pltpu.VMEM((KEPT_PAD,), jnp.float32),
            pltpu.VMEM((KEPT_PAD,), jnp.float32),
            pltpu.VMEM((KEPT_PAD,), jnp.float32),
            pltpu.VMEM((KEPT_PAD,), jnp.float32),
            pltpu.VMEM((KEPT_PAD,), jnp.float32),
            pltpu.VMEM((OUT_FLAT,), jnp.float32),
            pltpu.SemaphoreType.DMA,
            pltpu.SemaphoreType.DMA,
            pltpu.SemaphoreType.DMA,
        ],
    )
    out, _ = run(y_pred.reshape(BATCH, N_BOXES * FEAT))
    return out.reshape(BATCH, TOP_K, OUT_W)[:, :, :6]


# speedup vs baseline: 4.8734x; 1.1217x over previous
import jax
import jax.numpy as jnp
from jax import lax
from jax.experimental import pallas as pl
from jax.experimental.pallas import tpu as pltpu
from jax.experimental.pallas import tpu_sc as plsc

CLASS_NUM = 20
CONF_THRESH = 0.01
IOU_THRESHOLD = 0.45
TOP_K = 200
IMG_W = 512.0
BATCH = 8
N_BOXES = 20000
FEAT = 3 * CLASS_NUM + 5

N_PAD = 20480
NEG = -1e30

L = 16
NG1 = N_PAD // (L * L)
NG2 = NG1 // L
KEPT_PAD = 224
OUT_W = 8
OUT_FLAT = TOP_K * OUT_W

BPC = BATCH // 2
WPB = 4
DEC_PER_W = N_PAD // WPB
DEC_CH = 160
DEC_NCH = DEC_PER_W // DEC_CH
DEC_NCH_LAST = (N_BOXES - (WPB - 1) * DEC_PER_W) // DEC_CH
SFEAT = FEAT - CLASS_NUM
P2OFF = CLASS_NUM + 1
BOXOFF = 2 * CLASS_NUM + 1
ST0 = DEC_CH * SFEAT


def _sc_body(y_hbm, out_hbm, dec_hbm,
             conf_v, cls_v, x1_v, y1_v, x2_v, y2_v, l1_v, l2_v,
             kx1_v, ky1_v, kx2_v, ky2_v, kar_v, out_v, dsemA, dsemB, ssem):
    c = lax.axis_index("c")
    s = lax.axis_index("s")
    iota = jnp.arange(L, dtype=jnp.int32)

    lb = s // WPB
    q = s - lb * WPB
    b = BPC * c + lb
    n0 = q * DEC_PER_W
    nch = jnp.where(q == WPB - 1, DEC_NCH_LAST, DEC_NCH)

    CHW = DEC_CH * SFEAT

    def fire(ci, yref, base, sem):
        start = n0 + ci * DEC_CH
        pltpu.make_async_copy(y_hbm.at[b, pl.ds(start * SFEAT, CHW)],
                              yref.at[pl.ds(base, CHW)], sem).start()

    def compute(ci, yref, base):
        start = n0 + ci * DEC_CH

        def feat(rows, f):
            return plsc.load_gather(yref, [base + rows * SFEAT + f])

        for g in range(DEC_CH // L):
            rows = g * L + iota
            best = feat(rows, 0) * feat(rows, P2OFF)
            bid = jnp.zeros((L,), jnp.float32)
            for cc in range(1, CLASS_NUM):
                pc = (feat(rows, cc)
                      * feat(rows, P2OFF + cc))
                upd = pc > best
                best = jnp.where(upd, pc, best)
                bid = jnp.where(upd, jnp.float32(cc), bid)

            def clipv(v):
                return jnp.maximum(jnp.minimum(v, IMG_W - 1.0), 0.0)

            o = ST0 + g * L
            x1_v[pl.ds(o, L)] = best
            x1_v[pl.ds(o + DEC_CH, L)] = bid + 1.0
            x1_v[pl.ds(o + 2 * DEC_CH, L)] = clipv(feat(rows, BOXOFF))
            x1_v[pl.ds(o + 3 * DEC_CH, L)] = clipv(feat(rows, BOXOFF + 1))
            x1_v[pl.ds(o + 4 * DEC_CH, L)] = clipv(feat(rows, BOXOFF + 2))
            x1_v[pl.ds(o + 5 * DEC_CH, L)] = clipv(feat(rows, BOXOFF + 3))

        for a in range(6):
            pltpu.sync_copy(x1_v.at[pl.ds(ST0 + a * DEC_CH, DEC_CH)],
                            dec_hbm.at[a, b, pl.ds(start, DEC_CH)])

    fire(0, cls_v, 0, dsemA)

    def chunk(ci, _):
        def even():
            start = n0 + ci * DEC_CH
            pltpu.make_async_copy(y_hbm.at[b, pl.ds(start * SFEAT, CHW)],
                                  cls_v.at[pl.ds(0, CHW)], dsemA).wait()

            @pl.when(ci + 1 < nch)
            def _f():
                fire(ci + 1, y1_v, 0, dsemB)
            compute(ci, cls_v, 0)

        def odd():
            start = n0 + ci * DEC_CH
            pltpu.make_async_copy(y_hbm.at[b, pl.ds(start * SFEAT, CHW)],
                                  y1_v.at[pl.ds(0, CHW)], dsemB).wait()

            @pl.when(ci + 1 < nch)
            def _f():
                fire(ci + 1, cls_v, 0, dsemA)
            compute(ci, y1_v, 0)

        lax.cond(ci % 2 == 0, even, odd)
        return 0

    lax.fori_loop(0, nch, chunk, 0)
    plsc.subcore_barrier()

    @pl.when(s < BPC)
    def _run():
        gb = BPC * c + s
        arrs = (conf_v, cls_v, x1_v, y1_v, x2_v, y2_v)
        for a, dstv in enumerate(arrs):
            pltpu.make_async_copy(dec_hbm.at[a, gb], dstv, ssem).start()
        for a, dstv in enumerate(arrs):
            pltpu.make_async_copy(dec_hbm.at[a, gb], dstv, ssem).wait()

        zf = jnp.zeros((L,), jnp.float32)
        lane0 = iota == 0
        negv = jnp.full((L,), NEG, jnp.float32)

        for i in range((N_PAD - N_BOXES) // L):
            conf_v[pl.ds(N_BOXES + i * L, L)] = negv

        def _zero(i, _):
            out_v[pl.ds(i * L, L)] = zf
            return 0
        lax.fori_loop(0, OUT_FLAT // L, _zero, 0)
        for w in range(KEPT_PAD // L):
            kx1_v[pl.ds(w * L, L)] = zf
            ky1_v[pl.ds(w * L, L)] = zf
            kx2_v[pl.ds(w * L, L)] = zf
            ky2_v[pl.ds(w * L, L)] = zf
            kar_v[pl.ds(w * L, L)] = zf

        def _build_l1(g, _):
            acc = negv
            base = g * (L * L)
            for k in range(L):
                v = conf_v[pl.ds(base + k * L, L)]
                v = jnp.where(v > CONF_THRESH, v, NEG)
                conf_v[pl.ds(base + k * L, L)] = v
                acc = jnp.maximum(acc, v)
            l1_v[pl.ds(g * L, L)] = acc
            return 0
        lax.fori_loop(0, NG1, _build_l1, 0)

        for h in range(NG2):
            acc = negv
            for k in range(L):
                acc = jnp.maximum(acc, l1_v[pl.ds((h * L + k) * L, L)])
            l2_v[pl.ds(h * L, L)] = acc

        def top_max():
            acc = l2_v[pl.ds(0, L)]
            for h in range(1, NG2):
                acc = jnp.maximum(acc, l2_v[pl.ds(h * L, L)])
            return jnp.max(acc)

        def cond(st):
            kcnt, m = st
            return (kcnt < TOP_K) & (m > CONF_THRESH)

        def body(st):
            kcnt, m = st
            msp = jnp.full((L,), m)

            hbest = jnp.full((L,), NG2, jnp.int32)
            for h in range(NG2):
                vec = l2_v[pl.ds(h * L, L)]
                hbest = jnp.minimum(hbest, jnp.where(vec == msp, h, NG2))
            hstar = jnp.min(hbest)

            kbest = jnp.full((L,), L, jnp.int32)
            for k in range(L):
                vec = l1_v[pl.ds((hstar * L + k) * L, L)]
                kbest = jnp.minimum(kbest, jnp.where(vec == msp, k, L))
            gstar = hstar * L + jnp.min(kbest)

            kbest2 = jnp.full((L,), L, jnp.int32)
            for k in range(L):
                vec = conf_v[pl.ds((gstar * L + k) * L, L)]
                kbest2 = jnp.minimum(kbest2, jnp.where(vec == msp, k, L))
            istar = gstar * L + jnp.min(kbest2)

            civ = conf_v[pl.ds(istar * L, L)]
            jstar = jnp.min(jnp.where(civ == msp, iota, L))
            bstar = istar * L + jstar
            bsp = jnp.full((L,), bstar, jnp.int32)

            sx1 = plsc.load_gather(x1_v, [bsp])
            sy1 = plsc.load_gather(y1_v, [bsp])
            sx2 = plsc.load_gather(x2_v, [bsp])
            sy2 = plsc.load_gather(y2_v, [bsp])
            scl = plsc.load_gather(cls_v, [bsp])
            sar = (jnp.maximum(sx2 - sx1, 0.0) * jnp.maximum(sy2 - sy1, 0.0))

            supm = jnp.zeros((L,), jnp.bool_)
            for w in range(KEPT_PAD // L):
                kx1 = kx1_v[pl.ds(w * L, L)]
                ky1 = ky1_v[pl.ds(w * L, L)]
                kx2 = kx2_v[pl.ds(w * L, L)]
                ky2 = ky2_v[pl.ds(w * L, L)]
                kar = kar_v[pl.ds(w * L, L)]
                xx1 = jnp.maximum(kx1, sx1)
                yy1 = jnp.maximum(ky1, sy1)
                xx2 = jnp.minimum(kx2, sx2)
                yy2 = jnp.minimum(ky2, sy2)
                inter = (jnp.maximum(xx2 - xx1, 0.0)
                         * jnp.maximum(yy2 - yy1, 0.0))
                union = kar + sar - inter
                sv = (union > 0.0) & (inter > IOU_THRESHOLD
                                      * jnp.maximum(union, 1e-12))
                supm = supm | sv
            keepb = jnp.max(supm.astype(jnp.int32)) == 0

            ksp = jnp.full((L,), kcnt, jnp.int32)
            appm = lane0 & keepb
            plsc.store_scatter(kx1_v, [ksp], sx1, mask=appm)
            plsc.store_scatter(ky1_v, [ksp], sy1, mask=appm)
            plsc.store_scatter(kx2_v, [ksp], sx2, mask=appm)
            plsc.store_scatter(ky2_v, [ksp], sy2, mask=appm)
            plsc.store_scatter(kar_v, [ksp], sar, mask=appm)

            row = jnp.where(iota == 0, scl,
                  jnp.where(iota == 1, msp,
                  jnp.where(iota == 2, sx1,
                  jnp.where(iota == 3, sy1,
                  jnp.where(iota == 4, sx2, sy2)))))
            owm = (iota < 6) & keepb
            plsc.store_scatter(out_v, [ksp * OUT_W + iota], row, mask=owm)

            plsc.store_scatter(conf_v, [bsp], negv, mask=lane0)
            l1slot = gstar * L + jstar
            gv = plsc.load_gather(conf_v, [(gstar * L + iota) * L + jstar])
            plsc.store_scatter(l1_v, [jnp.full((L,), l1slot, jnp.int32)],
                               jnp.full((L,), jnp.max(gv)), mask=lane0)
            l2slot = hstar * L + jstar
            hv = plsc.load_gather(l1_v, [(hstar * L + iota) * L + jstar])
            plsc.store_scatter(l2_v, [jnp.full((L,), l2slot, jnp.int32)],
                               jnp.full((L,), jnp.max(hv)), mask=lane0)

            return (kcnt + jnp.where(keepb, 1, 0), top_max())

        lax.while_loop(cond, body, (jnp.int32(0), top_max()))
        pltpu.sync_copy(out_v, out_hbm.at[BPC * c + s])


@jax.jit
def kernel(y_pred):
    run = pl.kernel(
        _sc_body,
        out_type=[jax.ShapeDtypeStruct((BATCH, OUT_FLAT), jnp.float32),
                  jax.ShapeDtypeStruct((6, BATCH, N_PAD), jnp.float32)],
        mesh=plsc.VectorSubcoreMesh(core_axis_name="c", subcore_axis_name="s",
                                    num_cores=2, num_subcores=16),
        compiler_params=pltpu.CompilerParams(needs_layout_passes=False,
                                             use_tc_tiling_on_sc=False),
        scratch_types=[
            pltpu.VMEM((N_PAD,), jnp.float32),
            pltpu.VMEM((N_PAD,), jnp.float32),
            pltpu.VMEM((N_PAD,), jnp.float32),
            pltpu.VMEM((N_PAD,), jnp.float32),
            pltpu.VMEM((N_PAD,), jnp.float32),
            pltpu.VMEM((N_PAD,), jnp.float32),
            pltpu.VMEM((NG1 * L,), jnp.float32),
            pltpu.VMEM((NG2 * L,), jnp.float32),
            pltpu.VMEM((KEPT_PAD,), jnp.float32),
            pltpu.VMEM((KEPT_PAD,), jnp.float32),
            pltpu.VMEM((KEPT_PAD,), jnp.float32),
            pltpu.VMEM((KEPT_PAD,), jnp.float32),
            pltpu.VMEM((KEPT_PAD,), jnp.float32),
            pltpu.VMEM((OUT_FLAT,), jnp.float32),
            pltpu.SemaphoreType.DMA,
            pltpu.SemaphoreType.DMA,
            pltpu.SemaphoreType.DMA,
        ],
    )
    ysl = y_pred[:, :, CLASS_NUM:]
    out, _ = run(ysl.reshape(BATCH, N_BOXES * SFEAT))
    return out.reshape(BATCH, TOP_K, OUT_W)[:, :, :6]
